# baseline (speedup 1.0000x reference)
import functools

import jax
import jax.numpy as jnp
from jax import lax
from jax.experimental import pallas as pl
from jax.experimental.pallas import tpu as pltpu
from jax.experimental.pallas import tpu_sc as plsc

N = 10000
E = 320000
DIN = 128
DE = 16
DEA = 32
DH = 152
DHP = 160
DHH = DHP // 2
DIH = DIN // 2
DOUT = 128
NCLS = 15

NCORES = 2
NSUB = 16
NW = NCORES * NSUB
CH = 80
NCHT = E // (NSUB * CH)
NPAIR = NCHT // 2
EPT = E // NW
NCHP = EPT // CH
RPT = N // NSUB

RB = 1000
GRID = N // RB
SCW = 32

_mesh = plsc.VectorSubcoreMesh(core_axis_name="c", subcore_axis_name="s")
_params = pltpu.CompilerParams(use_tc_tiling_on_sc=False)


def _startg(tbl, idx, buf, sem):
    pltpu.async_copy(tbl.at[idx], buf, sem)


def _waitg(tbl, idx, buf, sem):
    pltpu.make_async_copy(tbl.at[idx], buf, sem).wait()


@functools.partial(
    pl.kernel,
    out_type=jax.ShapeDtypeStruct((NCORES, N, DIH), jnp.float32),
    mesh=_mesh,
    compiler_params=_params,
    scratch_types=(
        pltpu.VMEM_SHARED((N, DIH), jnp.float32),
        pltpu.VMEM((NCHT * CH,), jnp.int32),
        pltpu.VMEM((NCHT * CH,), jnp.int32),
        pltpu.VMEM((CH, DIH), jnp.float32),
        pltpu.VMEM((CH, DIH), jnp.float32),
        pltpu.SemaphoreType.DMA,
        pltpu.SemaphoreType.DMA,
    ),
)
def _sc_agg1(ei, nfl, nfr, zh, outh,
             acch, sidx, didx, r0b, r1b, g0, g1):
    cid = lax.axis_index("c")
    sid = lax.axis_index("s")
    row0 = sid * RPT
    pltpu.sync_copy(zh.at[pl.ds(row0, RPT)], acch.at[pl.ds(row0, RPT)])
    ebase = sid * NCHT * CH
    pltpu.sync_copy(ei.at[0, pl.ds(ebase, NCHT * CH)], sidx)
    pltpu.sync_copy(ei.at[1, pl.ds(ebase, NCHT * CH)], didx)
    plsc.subcore_barrier()

    def pipeline(tbl):
        def si(j):
            return sidx.at[pl.ds(j * CH, CH)]

        def do(j, rb, gs):
            _waitg(tbl, si(j), rb, gs)
            pltpu.sync_copy(rb, acch.at[didx.at[pl.ds(j * CH, CH)]], add=True)

        _startg(tbl, si(0), r0b, g0)

        @pl.loop(0, NPAIR)
        def _(p):
            j0 = 2 * p
            _startg(tbl, si(j0 + 1), r1b, g1)
            do(j0, r0b, g0)

            @pl.when(p < NPAIR - 1)
            def _():
                _startg(tbl, si(j0 + 2), r0b, g0)

            do(j0 + 1, r1b, g1)

    @pl.when(cid == 0)
    def _():
        pipeline(nfl)

    @pl.when(cid == 1)
    def _():
        pipeline(nfr)

    plsc.subcore_barrier()
    pltpu.sync_copy(acch.at[pl.ds(row0, RPT)], outh.at[cid, pl.ds(row0, RPT)])


@functools.partial(
    pl.kernel,
    out_type=(
        jax.ShapeDtypeStruct((NCORES, N, DE), jnp.float32),
        jax.ShapeDtypeStruct((NCORES, N, DE), jnp.float32),
    ),
    mesh=_mesh,
    compiler_params=_params,
    scratch_types=(
        pltpu.VMEM_SHARED((N, DE), jnp.float32),
        pltpu.VMEM_SHARED((N, DE), jnp.float32),
        pltpu.VMEM((NCHP * CH,), jnp.int32),
        pltpu.VMEM((CH, DE), jnp.float32),
        pltpu.VMEM((CH, DE), jnp.float32),
        pltpu.VMEM((CH, DE), jnp.float32),
        pltpu.SemaphoreType.DMA,
        pltpu.SemaphoreType.DMA,
    ),
)
def _sc_ec(ei, ef, on, ze, outse, outc,
           accse, accc, didx, e0b, e1b, ones, f0, f1):
    cid = lax.axis_index("c")
    sid = lax.axis_index("s")
    row0 = sid * RPT
    pltpu.sync_copy(ze.at[pl.ds(row0, RPT)], accse.at[pl.ds(row0, RPT)])
    pltpu.sync_copy(ze.at[pl.ds(row0, RPT)], accc.at[pl.ds(row0, RPT)])
    pltpu.sync_copy(on, ones)
    ebase = (cid * NSUB + sid) * EPT
    pltpu.sync_copy(ei.at[1, pl.ds(ebase, NCHP * CH)], didx)
    plsc.subcore_barrier()

    def starte(j, eb, sem):
        pltpu.async_copy(ef.at[pl.ds(ebase + j * CH, CH)], eb, sem)

    def do(j, eb, sem):
        pltpu.make_async_copy(
            ef.at[pl.ds(ebase + j * CH, CH)], eb, sem).wait()
        di = didx.at[pl.ds(j * CH, CH)]
        pltpu.sync_copy(eb, accse.at[di], add=True)
        pltpu.sync_copy(ones, accc.at[di], add=True)

    starte(0, e0b, f0)

    @pl.loop(0, (NCHP - 1) // 2)
    def _(p):
        j0 = 2 * p
        starte(j0 + 1, e1b, f1)
        do(j0, e0b, f0)
        starte(j0 + 2, e0b, f0)
        do(j0 + 1, e1b, f1)

    do(NCHP - 1, e0b, f0)

    plsc.subcore_barrier()
    pltpu.sync_copy(accse.at[pl.ds(row0, RPT)], outse.at[cid, pl.ds(row0, RPT)])
    pltpu.sync_copy(accc.at[pl.ds(row0, RPT)], outc.at[cid, pl.ds(row0, RPT)])


def _make_sc_agg(hw):
    @functools.partial(
        pl.kernel,
        out_type=jax.ShapeDtypeStruct((NCORES, N, hw), jnp.float32),
        mesh=_mesh,
        compiler_params=_params,
        scratch_types=(
            pltpu.VMEM_SHARED((N, hw), jnp.float32),
            pltpu.VMEM((NCHT * CH,), jnp.int32),
            pltpu.VMEM((NCHT * CH,), jnp.int32),
            pltpu.VMEM((CH, hw), jnp.float32),
            pltpu.VMEM((CH, hw), jnp.float32),
            pltpu.SemaphoreType.DMA,
            pltpu.SemaphoreType.DMA,
        ),
    )
    def sc_agg(ei, hl, hr, z, out, acc, sidx, didx, r0b, r1b, g0, g1):
        cid = lax.axis_index("c")
        sid = lax.axis_index("s")
        row0 = sid * RPT
        pltpu.sync_copy(z.at[pl.ds(row0, RPT)], acc.at[pl.ds(row0, RPT)])
        ebase = sid * NCHT * CH
        pltpu.sync_copy(ei.at[0, pl.ds(ebase, NCHT * CH)], sidx)
        pltpu.sync_copy(ei.at[1, pl.ds(ebase, NCHT * CH)], didx)
        plsc.subcore_barrier()

        def pipeline(tbl):
            def si(j):
                return sidx.at[pl.ds(j * CH, CH)]

            def do(j, rb, gs):
                _waitg(tbl, si(j), rb, gs)
                pltpu.sync_copy(rb, acc.at[didx.at[pl.ds(j * CH, CH)]],
                                add=True)

            _startg(tbl, si(0), r0b, g0)

            @pl.loop(0, NPAIR)
            def _(p):
                j0 = 2 * p
                _startg(tbl, si(j0 + 1), r1b, g1)
                do(j0, r0b, g0)

                @pl.when(p < NPAIR - 1)
                def _():
                    _startg(tbl, si(j0 + 2), r0b, g0)

                do(j0 + 1, r1b, g1)

        @pl.when(cid == 0)
        def _():
            pipeline(hl)

        @pl.when(cid == 1)
        def _():
            pipeline(hr)

        plsc.subcore_barrier()
        pltpu.sync_copy(acc.at[pl.ds(row0, RPT)],
                        out.at[cid, pl.ds(row0, RPT)])

    return sc_agg


_sc_agg_h = _make_sc_agg(DHH)
_sc_agg_y = _make_sc_agg(DOUT // 2)


@functools.partial(
    pl.kernel,
    out_type=jax.ShapeDtypeStruct((E, DE), jnp.float32),
    mesh=_mesh,
    compiler_params=_params,
    scratch_types=(
        pltpu.VMEM((NCHP * CH,), jnp.int32),
        pltpu.VMEM((NCHP * CH,), jnp.int32),
        pltpu.VMEM((CH, DE), jnp.float32),
        pltpu.VMEM((CH, DE), jnp.float32),
        pltpu.VMEM((CH, DE), jnp.float32),
        pltpu.VMEM((CH, DE), jnp.float32),
        pltpu.VMEM((CH, DE), jnp.float32),
        pltpu.VMEM((CH, DE), jnp.float32),
        pltpu.SemaphoreType.DMA,
        pltpu.SemaphoreType.DMA,
        pltpu.SemaphoreType.DMA,
        pltpu.SemaphoreType.DMA,
        pltpu.SemaphoreType.DMA,
        pltpu.SemaphoreType.DMA,
    ),
)
def _sc_pred(ei, pu, pv, out,
             sidx, didx, a0b, a1b, b0b, b1b, o0b, o1b,
             ga0, ga1, gb0, gb1, w0, w1):
    cid = lax.axis_index("c")
    sid = lax.axis_index("s")
    wid = cid * NSUB + sid
    ebase = wid * EPT
    pltpu.sync_copy(ei.at[0, pl.ds(ebase, NCHP * CH)], sidx)
    pltpu.sync_copy(ei.at[1, pl.ds(ebase, NCHP * CH)], didx)

    def startg(j, ab, bb, gsa, gsb):
        pltpu.async_copy(pu.at[sidx.at[pl.ds(j * CH, CH)]], ab, gsa)
        pltpu.async_copy(pv.at[didx.at[pl.ds(j * CH, CH)]], bb, gsb)

    def dst(j):
        return out.at[pl.ds(ebase + j * CH, CH)]

    def waitw(j, ob, ws):
        pltpu.make_async_copy(ob, dst(j), ws).wait()

    def do(j, jw, ab, bb, ob, gsa, gsb, ws, first):
        pltpu.make_async_copy(pu.at[sidx.at[pl.ds(j * CH, CH)]], ab, gsa).wait()
        pltpu.make_async_copy(pv.at[didx.at[pl.ds(j * CH, CH)]], bb, gsb).wait()
        if not first:
            waitw(jw, ob, ws)

        @pl.loop(0, CH, unroll=8)
        def _(i):
            ob[i, :] = ab[i, :] + bb[i, :]

        pltpu.async_copy(ob, dst(j), ws)

    startg(0, a0b, b0b, ga0, gb0)
    startg(1, a1b, b1b, ga1, gb1)
    do(0, 0, a0b, b0b, o0b, ga0, gb0, w0, True)
    startg(2, a0b, b0b, ga0, gb0)
    do(1, 0, a1b, b1b, o1b, ga1, gb1, w1, True)

    @pl.loop(1, (NCHP - 1) // 2)
    def _(p):
        j0 = 2 * p
        startg(j0 + 1, a1b, b1b, ga1, gb1)
        do(j0, j0 - 2, a0b, b0b, o0b, ga0, gb0, w0, False)
        startg(j0 + 2, a0b, b0b, ga0, gb0)
        do(j0 + 1, j0 - 1, a1b, b1b, o1b, ga1, gb1, w1, False)

    jl = NCHP - 1
    do(jl, jl - 2, a0b, b0b, o0b, ga0, gb0, w0, False)
    waitw(jl - 1, o1b, w1)
    waitw(jl, o0b, w0)


def _dot(a, b):
    return jnp.dot(a, b, preferred_element_type=jnp.float32)


def _relu_layer(p, se, cnt, rinv, hp, wmh, wme, bm, wah, wan, ba):
    neigh = (_dot(p, wmh[...]) + _dot(se, wme[...])
             + bm[...][None, :] * cnt) * rinv
    return jax.nn.relu(
        _dot(hp, wah[...]) + _dot(neigh, wan[...]) + ba[...][None, :])


def _split_h(h, hl_out, hr_out):
    hl_out[...] = h[:, :DHH]
    hr_out[...] = jnp.concatenate(
        [h[:, DHH:], jnp.zeros((RB, DHP - DH), jnp.float32)], axis=1)


def _tc1_body(nfl, nfr, aggh, aggse, aggc, wmh, wme, bm, wah, wan, ba,
              hl_out, hr_out, sc_out):
    p = jnp.concatenate([aggh[0], aggh[1]], axis=1)
    se = aggse[0] + aggse[1]
    cnt = (aggc[0] + aggc[1])[:, 0:1]
    rinv = 1.0 / jnp.maximum(cnt, 1.0)
    nf = jnp.concatenate([nfl[...], nfr[...]], axis=1)
    h = _relu_layer(p, se, cnt, rinv, nf, wmh, wme, bm, wah, wan, ba)
    _split_h(h, hl_out, hr_out)
    sc_out[...] = jnp.concatenate(
        [se, cnt, rinv, jnp.zeros((RB, SCW - DE - 2), jnp.float32)], axis=1)


def _tc_layer_body(hl, hr, agg, sc, wmh, wme, bm, wah, wan, ba, wnext,
                   hl_out, hr_out, yl_out, yr_out):
    p = jnp.concatenate([agg[0], agg[1]], axis=1)
    se = sc[:, 0:DE]
    cnt = sc[:, DE:DE + 1]
    rinv = sc[:, DE + 1:DE + 2]
    hp = jnp.concatenate([hl[...], hr[...]], axis=1)
    h = _relu_layer(p, se, cnt, rinv, hp, wmh, wme, bm, wah, wan, ba)
    _split_h(h, hl_out, hr_out)
    y = _dot(h, wnext[...])
    yl_out[...] = y[:, :DOUT // 2]
    yr_out[...] = y[:, DOUT // 2:]


def _tc3_body(hl, hr, agg, sc, wme, bm, wah, wan, ba, wpu, bp, wpv,
              pu_out, pv_out):
    pm = jnp.concatenate([agg[0], agg[1]], axis=1)
    se = sc[:, 0:DE]
    cnt = sc[:, DE:DE + 1]
    rinv = sc[:, DE + 1:DE + 2]
    neigh = (pm + _dot(se, wme[...]) + bm[...][None, :] * cnt) * rinv
    hp = jnp.concatenate([hl[...], hr[...]], axis=1)
    h = jax.nn.relu(
        _dot(hp, wah[...]) + _dot(neigh, wan[...]) + ba[...][None, :])
    pu_out[...] = _dot(h, wpu[...]) + bp[...][None, :]
    pv_out[...] = _dot(h, wpv[...])


def _row_spec(d):
    return pl.BlockSpec((RB, d), lambda i: (i, 0))


def _agg_spec(d):
    return pl.BlockSpec((NCORES, RB, d), lambda i: (0, i, 0))


def _full_spec(shape):
    nd = len(shape)
    return pl.BlockSpec(shape, lambda i, _nd=nd: (0,) * _nd)


def _tc1(nfl, nfr, aggh, aggse, aggc, wmh, wme, bm, wah, wan, ba):
    return pl.pallas_call(
        _tc1_body,
        grid=(GRID,),
        in_specs=[
            _row_spec(DIH), _row_spec(DIH), _agg_spec(DIH), _agg_spec(DE),
            _agg_spec(DE),
            _full_spec(wmh.shape), _full_spec(wme.shape), _full_spec(bm.shape),
            _full_spec(wah.shape), _full_spec(wan.shape), _full_spec(ba.shape),
        ],
        out_specs=[_row_spec(DHH), _row_spec(DHH), _row_spec(SCW)],
        out_shape=[
            jax.ShapeDtypeStruct((N, DHH), jnp.float32),
            jax.ShapeDtypeStruct((N, DHH), jnp.float32),
            jax.ShapeDtypeStruct((N, SCW), jnp.float32),
        ],
    )(nfl, nfr, aggh, aggse, aggc, wmh, wme, bm, wah, wan, ba)


def _tc_layer(hl, hr, agg, sc, wmh, wme, bm, wah, wan, ba, wnext):
    return pl.pallas_call(
        _tc_layer_body,
        grid=(GRID,),
        in_specs=[
            _row_spec(DHH), _row_spec(DHH), _agg_spec(DHH), _row_spec(SCW),
            _full_spec(wmh.shape), _full_spec(wme.shape), _full_spec(bm.shape),
            _full_spec(wah.shape), _full_spec(wan.shape), _full_spec(ba.shape),
            _full_spec(wnext.shape),
        ],
        out_specs=[_row_spec(DHH), _row_spec(DHH),
                   _row_spec(DOUT // 2), _row_spec(DOUT // 2)],
        out_shape=[
            jax.ShapeDtypeStruct((N, DHH), jnp.float32),
            jax.ShapeDtypeStruct((N, DHH), jnp.float32),
            jax.ShapeDtypeStruct((N, DOUT // 2), jnp.float32),
            jax.ShapeDtypeStruct((N, DOUT // 2), jnp.float32),
        ],
    )(hl, hr, agg, sc, wmh, wme, bm, wah, wan, ba, wnext)


def _tc3(hl, hr, agg, sc, wme, bm, wah, wan, ba, wpu, bp, wpv):
    return pl.pallas_call(
        _tc3_body,
        grid=(GRID,),
        in_specs=[
            _row_spec(DHH), _row_spec(DHH), _agg_spec(DOUT // 2),
            _row_spec(SCW),
            _full_spec(wme.shape), _full_spec(bm.shape),
            _full_spec(wah.shape), _full_spec(wan.shape), _full_spec(ba.shape),
            _full_spec(wpu.shape), _full_spec(bp.shape), _full_spec(wpv.shape),
        ],
        out_specs=[_row_spec(DE), _row_spec(DE)],
        out_shape=[
            jax.ShapeDtypeStruct((N, DE), jnp.float32),
            jax.ShapeDtypeStruct((N, DE), jnp.float32),
        ],
    )(hl, hr, agg, sc, wme, bm, wah, wan, ba, wpu, bp, wpv)


TB = 8000
TGRID = E // TB


def _tail_body(x_hbm, o, buf, sem):
    i = pl.program_id(0)
    pltpu.make_async_copy(
        x_hbm.at[pl.ds(i * TB, TB)], buf, sem).start()
    pltpu.make_async_copy(
        x_hbm.at[pl.ds(i * TB, TB)], buf, sem).wait()
    o[...] = buf[:, :NCLS]


def _tc_tail(x):
    return pl.pallas_call(
        _tail_body,
        grid=(TGRID,),
        in_specs=[pl.BlockSpec(memory_space=pltpu.HBM)],
        out_specs=pl.BlockSpec((TB, NCLS), lambda i: (i, 0)),
        out_shape=jax.ShapeDtypeStruct((E, NCLS), jnp.float32),
        scratch_shapes=[pltpu.VMEM((TB, DE), jnp.float32),
                        pltpu.SemaphoreType.DMA],
    )(x)


def _padr(w, rows):
    return jnp.pad(w, ((0, rows - w.shape[0]), (0, 0)))


def kernel(nfeats, efeats, edge_index, Wm1, bm1, Wa1, ba1, Wm2, bm2, Wa2, ba2,
           Wm3, bm3, Wa3, ba3, Wp, bp):
    nf = nfeats.reshape(N, DIN)
    nfl = nf[:, :DIH]
    nfr = nf[:, DIH:]
    ei = edge_index
    on = jnp.ones((CH, DE), jnp.float32)
    zh = jnp.zeros((N, DIH), jnp.float32)
    ze = jnp.zeros((N, DE), jnp.float32)
    z80 = jnp.zeros((N, DHH), jnp.float32)
    z64 = jnp.zeros((N, DOUT // 2), jnp.float32)

    aggh = _sc_agg1(ei, nfl, nfr, zh)
    aggse, aggc = _sc_ec(ei, efeats.reshape(E, DE), on, ze)

    h1l, h1r, sc = _tc1(
        nfl, nfr, aggh, aggse, aggc,
        Wm1[:, :DIN].T, Wm1[:, DIN:].T, bm1,
        Wa1[:, :DIN].T, Wa1[:, DIN:].T, ba1)

    agg2 = _sc_agg_h(ei, h1l, h1r, z80)
    h2l, h2r, y3l, y3r = _tc_layer(
        h1l, h1r, agg2, sc,
        _padr(Wm2[:, :DH].T, DHP), Wm2[:, DH:].T, bm2,
        _padr(Wa2[:, :DH].T, DHP), Wa2[:, DH:].T, ba2,
        Wm3[:, :DH].T)

    agg3 = _sc_agg_y(ei, y3l, y3r, z64)
    wpu = jnp.pad(Wp[:, :DOUT].T, ((0, 0), (0, DE - NCLS)))
    wpv = jnp.pad(Wp[:, DOUT:].T, ((0, 0), (0, DE - NCLS)))
    bp16 = jnp.pad(bp, (0, DE - NCLS))
    pu, pv = _tc3(
        h2l, h2r, agg3, sc,
        Wm3[:, DH:].T, bm3,
        _padr(Wa3[:, :DH].T, DHP), Wa3[:, DH:].T, ba3,
        wpu, bp16, wpv)

    return _tc_tail(_sc_pred(ei, pu, pv))


# speedup vs baseline: 1.2059x; 1.2059x over previous
import functools

import jax
import jax.numpy as jnp
from jax import lax
from jax.experimental import pallas as pl
from jax.experimental.pallas import tpu as pltpu
from jax.experimental.pallas import tpu_sc as plsc

N = 10000
E = 320000
DIN = 128
DE = 16
DEA = 32
DH = 152
DHP = 160
DHH = DHP // 2
DIH = DIN // 2
DOUT = 128
NCLS = 15

NCORES = 2
NSUB = 16
NW = NCORES * NSUB
CH = 80
NCHT = E // (NSUB * CH)
NPAIR = NCHT // 2
EPT = E // NW
NCHP = EPT // CH
RPT = N // NSUB

RB = 1000
GRID = N // RB
SCW = 32

_mesh = plsc.VectorSubcoreMesh(core_axis_name="c", subcore_axis_name="s")
_params = pltpu.CompilerParams(use_tc_tiling_on_sc=False)


def _startg(tbl, idx, buf, sem):
    pltpu.async_copy(tbl.at[idx], buf, sem)


def _waitg(tbl, idx, buf, sem):
    pltpu.make_async_copy(tbl.at[idx], buf, sem).wait()


@functools.partial(
    pl.kernel,
    out_type=jax.ShapeDtypeStruct((NCORES, N, DIH), jnp.float32),
    mesh=_mesh,
    compiler_params=_params,
    scratch_types=(
        pltpu.VMEM_SHARED((N, DIH), jnp.float32),
        pltpu.VMEM((NCHT * CH,), jnp.int32),
        pltpu.VMEM((NCHT * CH,), jnp.int32),
        pltpu.VMEM((CH, DIH), jnp.float32),
        pltpu.VMEM((CH, DIH), jnp.float32),
        pltpu.SemaphoreType.DMA,
        pltpu.SemaphoreType.DMA,
    ),
)
def _sc_agg1(ei, nfl, nfr, zh, outh,
             acch, sidx, didx, r0b, r1b, g0, g1):
    cid = lax.axis_index("c")
    sid = lax.axis_index("s")
    row0 = sid * RPT
    pltpu.sync_copy(zh.at[pl.ds(row0, RPT)], acch.at[pl.ds(row0, RPT)])
    ebase = sid * NCHT * CH
    pltpu.sync_copy(ei.at[0, pl.ds(ebase, NCHT * CH)], sidx)
    pltpu.sync_copy(ei.at[1, pl.ds(ebase, NCHT * CH)], didx)
    plsc.subcore_barrier()

    def pipeline(tbl):
        def si(j):
            return sidx.at[pl.ds(j * CH, CH)]

        def do(j, rb, gs):
            _waitg(tbl, si(j), rb, gs)
            pltpu.sync_copy(rb, acch.at[didx.at[pl.ds(j * CH, CH)]], add=True)

        _startg(tbl, si(0), r0b, g0)

        @pl.loop(0, NPAIR)
        def _(p):
            j0 = 2 * p
            _startg(tbl, si(j0 + 1), r1b, g1)
            do(j0, r0b, g0)

            @pl.when(p < NPAIR - 1)
            def _():
                _startg(tbl, si(j0 + 2), r0b, g0)

            do(j0 + 1, r1b, g1)

    @pl.when(cid == 0)
    def _():
        pipeline(nfl)

    @pl.when(cid == 1)
    def _():
        pipeline(nfr)

    plsc.subcore_barrier()
    pltpu.sync_copy(acch.at[pl.ds(row0, RPT)], outh.at[cid, pl.ds(row0, RPT)])


@functools.partial(
    pl.kernel,
    out_type=(
        jax.ShapeDtypeStruct((NCORES, N, DE), jnp.float32),
        jax.ShapeDtypeStruct((NCORES, N, DE), jnp.float32),
    ),
    mesh=_mesh,
    compiler_params=_params,
    scratch_types=(
        pltpu.VMEM_SHARED((N, DE), jnp.float32),
        pltpu.VMEM_SHARED((N, DE), jnp.float32),
        pltpu.VMEM((NCHP * CH,), jnp.int32),
        pltpu.VMEM((CH, DE), jnp.float32),
        pltpu.VMEM((CH, DE), jnp.float32),
        pltpu.VMEM((CH, DE), jnp.float32),
        pltpu.SemaphoreType.DMA,
        pltpu.SemaphoreType.DMA,
    ),
)
def _sc_ec(ei, ef, on, ze, outse, outc,
           accse, accc, didx, e0b, e1b, ones, f0, f1):
    cid = lax.axis_index("c")
    sid = lax.axis_index("s")
    row0 = sid * RPT
    pltpu.sync_copy(ze.at[pl.ds(row0, RPT)], accse.at[pl.ds(row0, RPT)])
    pltpu.sync_copy(ze.at[pl.ds(row0, RPT)], accc.at[pl.ds(row0, RPT)])
    pltpu.sync_copy(on, ones)
    ebase = (cid * NSUB + sid) * EPT
    pltpu.sync_copy(ei.at[1, pl.ds(ebase, NCHP * CH)], didx)
    plsc.subcore_barrier()

    def starte(j, eb, sem):
        pltpu.async_copy(ef.at[pl.ds(ebase + j * CH, CH)], eb, sem)

    def do(j, eb, sem):
        pltpu.make_async_copy(
            ef.at[pl.ds(ebase + j * CH, CH)], eb, sem).wait()
        di = didx.at[pl.ds(j * CH, CH)]
        pltpu.sync_copy(eb, accse.at[di], add=True)
        pltpu.sync_copy(ones, accc.at[di], add=True)

    starte(0, e0b, f0)

    @pl.loop(0, (NCHP - 1) // 2)
    def _(p):
        j0 = 2 * p
        starte(j0 + 1, e1b, f1)
        do(j0, e0b, f0)
        starte(j0 + 2, e0b, f0)
        do(j0 + 1, e1b, f1)

    do(NCHP - 1, e0b, f0)

    plsc.subcore_barrier()
    pltpu.sync_copy(accse.at[pl.ds(row0, RPT)], outse.at[cid, pl.ds(row0, RPT)])
    pltpu.sync_copy(accc.at[pl.ds(row0, RPT)], outc.at[cid, pl.ds(row0, RPT)])


def _make_sc_agg(hw):
    @functools.partial(
        pl.kernel,
        out_type=jax.ShapeDtypeStruct((NCORES, N, hw), jnp.float32),
        mesh=_mesh,
        compiler_params=_params,
        scratch_types=(
            pltpu.VMEM_SHARED((N, hw), jnp.float32),
            pltpu.VMEM((NCHT * CH,), jnp.int32),
            pltpu.VMEM((NCHT * CH,), jnp.int32),
            pltpu.VMEM((CH, hw), jnp.float32),
            pltpu.VMEM((CH, hw), jnp.float32),
            pltpu.SemaphoreType.DMA,
            pltpu.SemaphoreType.DMA,
        ),
    )
    def sc_agg(ei, hl, hr, z, out, acc, sidx, didx, r0b, r1b, g0, g1):
        cid = lax.axis_index("c")
        sid = lax.axis_index("s")
        row0 = sid * RPT
        pltpu.sync_copy(z.at[pl.ds(row0, RPT)], acc.at[pl.ds(row0, RPT)])
        ebase = sid * NCHT * CH
        pltpu.sync_copy(ei.at[0, pl.ds(ebase, NCHT * CH)], sidx)
        pltpu.sync_copy(ei.at[1, pl.ds(ebase, NCHT * CH)], didx)
        plsc.subcore_barrier()

        def pipeline(tbl):
            def si(j):
                return sidx.at[pl.ds(j * CH, CH)]

            def do(j, rb, gs):
                _waitg(tbl, si(j), rb, gs)
                pltpu.sync_copy(rb, acc.at[didx.at[pl.ds(j * CH, CH)]],
                                add=True)

            _startg(tbl, si(0), r0b, g0)

            @pl.loop(0, NPAIR)
            def _(p):
                j0 = 2 * p
                _startg(tbl, si(j0 + 1), r1b, g1)
                do(j0, r0b, g0)

                @pl.when(p < NPAIR - 1)
                def _():
                    _startg(tbl, si(j0 + 2), r0b, g0)

                do(j0 + 1, r1b, g1)

        @pl.when(cid == 0)
        def _():
            pipeline(hl)

        @pl.when(cid == 1)
        def _():
            pipeline(hr)

        plsc.subcore_barrier()
        pltpu.sync_copy(acc.at[pl.ds(row0, RPT)],
                        out.at[cid, pl.ds(row0, RPT)])

    return sc_agg


_sc_agg_h = _make_sc_agg(DHH)
_sc_agg_y = _make_sc_agg(DOUT // 2)


@functools.partial(
    pl.kernel,
    out_type=jax.ShapeDtypeStruct((E, DE), jnp.float32),
    mesh=_mesh,
    compiler_params=_params,
    scratch_types=(
        pltpu.VMEM((NCHP * CH,), jnp.int32),
        pltpu.VMEM((NCHP * CH,), jnp.int32),
        pltpu.VMEM((CH, DE), jnp.float32),
        pltpu.VMEM((CH, DE), jnp.float32),
        pltpu.VMEM((CH, DE), jnp.float32),
        pltpu.VMEM((CH, DE), jnp.float32),
        pltpu.VMEM((CH, DE), jnp.float32),
        pltpu.VMEM((CH, DE), jnp.float32),
        pltpu.SemaphoreType.DMA,
        pltpu.SemaphoreType.DMA,
        pltpu.SemaphoreType.DMA,
        pltpu.SemaphoreType.DMA,
        pltpu.SemaphoreType.DMA,
        pltpu.SemaphoreType.DMA,
    ),
)
def _sc_pred(ei, pu, pv, out,
             sidx, didx, a0b, a1b, b0b, b1b, o0b, o1b,
             ga0, ga1, gb0, gb1, w0, w1):
    cid = lax.axis_index("c")
    sid = lax.axis_index("s")
    wid = cid * NSUB + sid
    ebase = wid * EPT
    pltpu.sync_copy(ei.at[0, pl.ds(ebase, NCHP * CH)], sidx)
    pltpu.sync_copy(ei.at[1, pl.ds(ebase, NCHP * CH)], didx)

    def startg(j, ab, bb, gsa, gsb):
        pltpu.async_copy(pu.at[sidx.at[pl.ds(j * CH, CH)]], ab, gsa)
        pltpu.async_copy(pv.at[didx.at[pl.ds(j * CH, CH)]], bb, gsb)

    def dst(j):
        return out.at[pl.ds(ebase + j * CH, CH)]

    def waitw(j, ob, ws):
        pltpu.make_async_copy(ob, dst(j), ws).wait()

    def do(j, jw, ab, bb, ob, gsa, gsb, ws, first):
        pltpu.make_async_copy(pu.at[sidx.at[pl.ds(j * CH, CH)]], ab, gsa).wait()
        pltpu.make_async_copy(pv.at[didx.at[pl.ds(j * CH, CH)]], bb, gsb).wait()
        if not first:
            waitw(jw, ob, ws)

        @pl.loop(0, CH, unroll=8)
        def _(i):
            ob[i, :] = ab[i, :] + bb[i, :]

        pltpu.async_copy(ob, dst(j), ws)

    startg(0, a0b, b0b, ga0, gb0)
    startg(1, a1b, b1b, ga1, gb1)
    do(0, 0, a0b, b0b, o0b, ga0, gb0, w0, True)
    startg(2, a0b, b0b, ga0, gb0)
    do(1, 0, a1b, b1b, o1b, ga1, gb1, w1, True)

    @pl.loop(1, (NCHP - 1) // 2)
    def _(p):
        j0 = 2 * p
        startg(j0 + 1, a1b, b1b, ga1, gb1)
        do(j0, j0 - 2, a0b, b0b, o0b, ga0, gb0, w0, False)
        startg(j0 + 2, a0b, b0b, ga0, gb0)
        do(j0 + 1, j0 - 1, a1b, b1b, o1b, ga1, gb1, w1, False)

    jl = NCHP - 1
    do(jl, jl - 2, a0b, b0b, o0b, ga0, gb0, w0, False)
    waitw(jl - 1, o1b, w1)
    waitw(jl, o0b, w0)


def _dot(a, b):
    return jnp.dot(a, b, preferred_element_type=jnp.float32)


def _relu_layer(p, se, cnt, rinv, hp, wmh, wme, bm, wah, wan, ba):
    neigh = (_dot(p, wmh[...]) + _dot(se, wme[...])
             + bm[...][None, :] * cnt) * rinv
    return jax.nn.relu(
        _dot(hp, wah[...]) + _dot(neigh, wan[...]) + ba[...][None, :])


def _split_h(h, hl_out, hr_out):
    hl_out[...] = h[:, :DHH]
    hr_out[...] = jnp.concatenate(
        [h[:, DHH:], jnp.zeros((RB, DHP - DH), jnp.float32)], axis=1)


def _tc1_body(nfl, nfr, aggh, aggse, aggc, wmh, wme, bm, wah, wan, ba,
              hl_out, hr_out, sc_out):
    p = jnp.concatenate([aggh[0], aggh[1]], axis=1)
    se = aggse[0] + aggse[1]
    cnt = (aggc[0] + aggc[1])[:, 0:1]
    rinv = 1.0 / jnp.maximum(cnt, 1.0)
    nf = jnp.concatenate([nfl[...], nfr[...]], axis=1)
    h = _relu_layer(p, se, cnt, rinv, nf, wmh, wme, bm, wah, wan, ba)
    _split_h(h, hl_out, hr_out)
    sc_out[...] = jnp.concatenate(
        [se, cnt, rinv, jnp.zeros((RB, SCW - DE - 2), jnp.float32)], axis=1)


def _tc_layer_body(hl, hr, agg, sc, wmh, wme, bm, wah, wan, ba, wnext,
                   hl_out, hr_out, yl_out, yr_out):
    p = jnp.concatenate([agg[0], agg[1]], axis=1)
    se = sc[:, 0:DE]
    cnt = sc[:, DE:DE + 1]
    rinv = sc[:, DE + 1:DE + 2]
    hp = jnp.concatenate([hl[...], hr[...]], axis=1)
    h = _relu_layer(p, se, cnt, rinv, hp, wmh, wme, bm, wah, wan, ba)
    _split_h(h, hl_out, hr_out)
    y = _dot(h, wnext[...])
    yl_out[...] = y[:, :DOUT // 2]
    yr_out[...] = y[:, DOUT // 2:]


def _tc3_body(hl, hr, agg, sc, wme, bm, wah, wan, ba, wpu, bp, wpv,
              pu_out, pv_out):
    pm = jnp.concatenate([agg[0], agg[1]], axis=1)
    se = sc[:, 0:DE]
    cnt = sc[:, DE:DE + 1]
    rinv = sc[:, DE + 1:DE + 2]
    neigh = (pm + _dot(se, wme[...]) + bm[...][None, :] * cnt) * rinv
    hp = jnp.concatenate([hl[...], hr[...]], axis=1)
    h = jax.nn.relu(
        _dot(hp, wah[...]) + _dot(neigh, wan[...]) + ba[...][None, :])
    pu_out[...] = _dot(h, wpu[...]) + bp[...][None, :]
    pv_out[...] = _dot(h, wpv[...])


def _row_spec(d):
    return pl.BlockSpec((RB, d), lambda i: (i, 0))


def _agg_spec(d):
    return pl.BlockSpec((NCORES, RB, d), lambda i: (0, i, 0))


def _full_spec(shape):
    nd = len(shape)
    return pl.BlockSpec(shape, lambda i, _nd=nd: (0,) * _nd)


def _tc1(nfl, nfr, aggh, aggse, aggc, wmh, wme, bm, wah, wan, ba):
    return pl.pallas_call(
        _tc1_body,
        grid=(GRID,),
        in_specs=[
            _row_spec(DIH), _row_spec(DIH), _agg_spec(DIH), _agg_spec(DE),
            _agg_spec(DE),
            _full_spec(wmh.shape), _full_spec(wme.shape), _full_spec(bm.shape),
            _full_spec(wah.shape), _full_spec(wan.shape), _full_spec(ba.shape),
        ],
        out_specs=[_row_spec(DHH), _row_spec(DHH), _row_spec(SCW)],
        out_shape=[
            jax.ShapeDtypeStruct((N, DHH), jnp.float32),
            jax.ShapeDtypeStruct((N, DHH), jnp.float32),
            jax.ShapeDtypeStruct((N, SCW), jnp.float32),
        ],
    )(nfl, nfr, aggh, aggse, aggc, wmh, wme, bm, wah, wan, ba)


def _tc_layer(hl, hr, agg, sc, wmh, wme, bm, wah, wan, ba, wnext):
    return pl.pallas_call(
        _tc_layer_body,
        grid=(GRID,),
        in_specs=[
            _row_spec(DHH), _row_spec(DHH), _agg_spec(DHH), _row_spec(SCW),
            _full_spec(wmh.shape), _full_spec(wme.shape), _full_spec(bm.shape),
            _full_spec(wah.shape), _full_spec(wan.shape), _full_spec(ba.shape),
            _full_spec(wnext.shape),
        ],
        out_specs=[_row_spec(DHH), _row_spec(DHH),
                   _row_spec(DOUT // 2), _row_spec(DOUT // 2)],
        out_shape=[
            jax.ShapeDtypeStruct((N, DHH), jnp.float32),
            jax.ShapeDtypeStruct((N, DHH), jnp.float32),
            jax.ShapeDtypeStruct((N, DOUT // 2), jnp.float32),
            jax.ShapeDtypeStruct((N, DOUT // 2), jnp.float32),
        ],
    )(hl, hr, agg, sc, wmh, wme, bm, wah, wan, ba, wnext)


def _tc3(hl, hr, agg, sc, wme, bm, wah, wan, ba, wpu, bp, wpv):
    return pl.pallas_call(
        _tc3_body,
        grid=(GRID,),
        in_specs=[
            _row_spec(DHH), _row_spec(DHH), _agg_spec(DOUT // 2),
            _row_spec(SCW),
            _full_spec(wme.shape), _full_spec(bm.shape),
            _full_spec(wah.shape), _full_spec(wan.shape), _full_spec(ba.shape),
            _full_spec(wpu.shape), _full_spec(bp.shape), _full_spec(wpv.shape),
        ],
        out_specs=[_row_spec(DE), _row_spec(DE)],
        out_shape=[
            jax.ShapeDtypeStruct((N, DE), jnp.float32),
            jax.ShapeDtypeStruct((N, DE), jnp.float32),
        ],
    )(hl, hr, agg, sc, wme, bm, wah, wan, ba, wpu, bp, wpv)


def _padr(w, rows):
    return jnp.pad(w, ((0, rows - w.shape[0]), (0, 0)))


def kernel(nfeats, efeats, edge_index, Wm1, bm1, Wa1, ba1, Wm2, bm2, Wa2, ba2,
           Wm3, bm3, Wa3, ba3, Wp, bp):
    nf = nfeats.reshape(N, DIN)
    nfl = nf[:, :DIH]
    nfr = nf[:, DIH:]
    ei = edge_index
    on = jnp.ones((CH, DE), jnp.float32)
    zh = jnp.zeros((N, DIH), jnp.float32)
    ze = jnp.zeros((N, DE), jnp.float32)
    z80 = jnp.zeros((N, DHH), jnp.float32)
    z64 = jnp.zeros((N, DOUT // 2), jnp.float32)

    aggh = _sc_agg1(ei, nfl, nfr, zh)
    aggse, aggc = _sc_ec(ei, efeats.reshape(E, DE), on, ze)

    h1l, h1r, sc = _tc1(
        nfl, nfr, aggh, aggse, aggc,
        Wm1[:, :DIN].T, Wm1[:, DIN:].T, bm1,
        Wa1[:, :DIN].T, Wa1[:, DIN:].T, ba1)

    agg2 = _sc_agg_h(ei, h1l, h1r, z80)
    h2l, h2r, y3l, y3r = _tc_layer(
        h1l, h1r, agg2, sc,
        _padr(Wm2[:, :DH].T, DHP), Wm2[:, DH:].T, bm2,
        _padr(Wa2[:, :DH].T, DHP), Wa2[:, DH:].T, ba2,
        Wm3[:, :DH].T)

    agg3 = _sc_agg_y(ei, y3l, y3r, z64)
    wpu = jnp.pad(Wp[:, :DOUT].T, ((0, 0), (0, DE - NCLS)))
    wpv = jnp.pad(Wp[:, DOUT:].T, ((0, 0), (0, DE - NCLS)))
    bp16 = jnp.pad(bp, (0, DE - NCLS))
    pu, pv = _tc3(
        h2l, h2r, agg3, sc,
        Wm3[:, DH:].T, bm3,
        _padr(Wa3[:, :DH].T, DHP), Wa3[:, DH:].T, ba3,
        wpu, bp16, wpv)

    score = _sc_pred(ei, pu, pv)
    return score[:, :NCLS]
